# Initial kernel scaffold; baseline (speedup 1.0000x reference)
import jax
import jax.numpy as jnp
from jax.experimental import pallas as pl


def kernel(L_indices, L_values, x, W, b):
    raise NotImplementedError("write your pallas kernel here")


# speedup vs baseline: 4.4157x; 4.4157x over previous
import functools

import jax
import jax.numpy as jnp
from jax import lax
from jax.experimental import pallas as pl
from jax.experimental.pallas import tpu as pltpu
from jax.experimental.pallas import tpu_sc as plsc

N = 10000
E = 320000
D = 128

NC = 2
NS = 16
EPT = E // (NC * NS)
G = 80
NGROUPS = EPT // G
SUB = 16
NSUB = G // SUB
NPAD = 10240
RPT = NPAD // NS
ZR = 128
LANES = 16


def _sc_segment_spmm(row, col, val, x):
    mesh = plsc.VectorSubcoreMesh(core_axis_name="c", subcore_axis_name="s")

    @functools.partial(
        pl.kernel,
        mesh=mesh,
        out_type=jax.ShapeDtypeStruct((NC * NPAD, D), jnp.float32),
        scratch_types=[
            pltpu.VMEM((G,), jnp.int32),
            pltpu.VMEM((G,), jnp.int32),
            pltpu.VMEM((G,), jnp.float32),
            pltpu.VMEM((G, D), jnp.float32),
            pltpu.VMEM((ZR, D), jnp.float32),
            pltpu.VMEM_SHARED((NPAD, D), jnp.float32),
            pltpu.SemaphoreType.DMA,
        ],
    )
    def k(row_h, col_h, val_h, x_h, out_h, colv, rowv, valv, xrows, zbuf,
          agg, sem):
        c = lax.axis_index("c")
        s = lax.axis_index("s")

        def zb(i, carry):
            for kk in range(D // LANES):
                zbuf[i, pl.ds(LANES * kk, LANES)] = jnp.zeros(
                    (LANES,), jnp.float32)
            return carry
        lax.fori_loop(0, ZR, zb, 0)
        for j in range(RPT // ZR):
            pltpu.sync_copy(zbuf, agg.at[pl.ds(s * RPT + j * ZR, ZR)])
        plsc.subcore_barrier()

        base0 = c * (NS * EPT) + s * EPT

        def group(g, carry):
            base = base0 + g * G
            pltpu.sync_copy(col_h.at[pl.ds(base, G)], colv)
            pltpu.sync_copy(val_h.at[pl.ds(base, G)], valv)
            pltpu.sync_copy(row_h.at[pl.ds(base, G)], rowv)
            pltpu.async_copy(x_h.at[colv], xrows, sem).wait()

            dnums = lax.GatherDimensionNumbers(
                offset_dims=(), collapsed_slice_dims=(0,),
                start_index_map=(0,))

            def sub(t, carry2):
                v16 = valv[pl.ds(t * SUB, SUB)]
                for e in range(SUB):
                    idx = t * SUB + e
                    vv = lax.gather(
                        v16, jnp.full((LANES, 1), e, jnp.int32), dnums, (1,),
                        mode=lax.GatherScatterMode.PROMISE_IN_BOUNDS)
                    for kk in range(D // LANES):
                        sl = pl.ds(LANES * kk, LANES)
                        xrows[idx, sl] = xrows[idx, sl] * vv
                return carry2
            lax.fori_loop(0, NSUB, sub, 0)

            pltpu.sync_copy(xrows, agg.at[rowv], add=True)
            return carry
        lax.fori_loop(0, NGROUPS, group, 0)

        plsc.subcore_barrier()
        pltpu.sync_copy(agg.at[pl.ds(s * RPT, RPT)],
                        out_h.at[pl.ds(c * NPAD + s * RPT, RPT)])

    return k(row, col, val, x)


def _tc_body(p0, p1, wt, bb, o):
    acc = p0[...] + p1[...]
    o[...] = jnp.dot(acc, wt[...],
                     preferred_element_type=jnp.float32) + bb[...]


def _tc_linear(partials, wt, b2):
    R = 1000
    return pl.pallas_call(
        _tc_body,
        grid=(N // R,),
        in_specs=[
            pl.BlockSpec((R, D), lambda i: (i, 0)),
            pl.BlockSpec((R, D), lambda i: (i + N // R, 0)),
            pl.BlockSpec((D, D), lambda i: (0, 0)),
            pl.BlockSpec((1, D), lambda i: (0, 0)),
        ],
        out_specs=pl.BlockSpec((R, D), lambda i: (i, 0)),
        out_shape=jax.ShapeDtypeStruct((N, D), jnp.float32),
    )(partials, partials, wt, b2)


def kernel(L_indices, L_values, x, W, b):
    row = L_indices[0]
    col = L_indices[1]
    pp = _sc_segment_spmm(row, col, L_values, x)
    partials = jnp.concatenate([pp[:N], pp[NPAD:NPAD + N]], axis=0)
    return _tc_linear(partials, W.T, b.reshape(1, D))
